# baseline (speedup 1.0000x reference)
import functools

import jax
import jax.numpy as jnp
from jax import lax
from jax.experimental import pallas as pl
from jax.experimental.pallas import tpu as pltpu
from jax.experimental.pallas import tpu_sc as plsc

NC = 2
NS = 16
LANES = 16
CH = 64


def _mm_scale_body(h_ref, w_ref, dinv_ref, o_ref):
    o_ref[...] = lax.dot_general(
        h_ref[...], w_ref[...], (((1,), (1,)), ((), ())),
        preferred_element_type=jnp.float32) * dinv_ref[...]


def _bn_relu(p0, p1, hs, dinv, b, g, bt):
    agg = (p0 + p1 + hs) * dinv + b
    mean = jnp.mean(agg, axis=0, keepdims=True)
    ctr = agg - mean
    var = jnp.mean(ctr * ctr, axis=0, keepdims=True)
    return jnp.maximum(g * ctr * lax.rsqrt(var + 1e-5) + bt, 0.0)


def _bn_mm_body(p0_ref, p1_ref, hs_ref, dinv_ref, b_ref, g_ref, bt_ref,
                w_ref, o_ref):
    h = _bn_relu(p0_ref[...], p1_ref[...], hs_ref[...], dinv_ref[...],
                 b_ref[...], g_ref[...], bt_ref[...])
    o_ref[...] = lax.dot_general(
        h, w_ref[...], (((1,), (1,)), ((), ())),
        preferred_element_type=jnp.float32) * dinv_ref[...]


def _dinv_body(p_ref, o_ref):
    deg = jnp.sum(p_ref[...], axis=0) + 1.0
    o_ref[...] = lax.rsqrt(deg)


def _bn_pool_body(p0_ref, p1_ref, hs_ref, dinv_ref, b_ref, g_ref, bt_ref,
                  g1w_ref, g1b_ref, g2wb_ref, wc_ref, bc_ref, o_ref):
    h = _bn_relu(p0_ref[...], p1_ref[...], hs_ref[...], dinv_ref[...],
                 b_ref[...], g_ref[...], bt_ref[...])
    z = jnp.maximum(
        lax.dot_general(h, g1w_ref[...], (((1,), (1,)), ((), ())),
                        preferred_element_type=jnp.float32) + g1b_ref[...], 0.0)
    gate = lax.dot_general(z, g2wb_ref[...], (((1,), (1,)), ((), ())),
                           preferred_element_type=jnp.float32)
    m = jnp.max(gate, axis=0, keepdims=True)
    e = jnp.exp(gate - m)
    alpha = e / jnp.sum(e, axis=0, keepdims=True)
    pooled = jnp.sum(alpha * h, axis=0, keepdims=True)
    o_ref[...] = lax.dot_general(pooled, wc_ref[...], (((1,), (1,)), ((), ())),
                                 preferred_element_type=jnp.float32) + bc_ref[...]


def _tc(body, shape, *args):
    return pl.pallas_call(
        body, out_shape=jax.ShapeDtypeStruct(shape, jnp.float32))(*args)


def _rows_per_tile(n):
    return -(-n // (NS * 8)) * 8


@functools.lru_cache(maxsize=None)
def _make_deg_kernel(n, ept):
    mesh = plsc.VectorSubcoreMesh(core_axis_name="c", subcore_axis_name="s",
                                  num_cores=NC, num_subcores=NS)
    hrows = -(-n // (128 * 8)) * 8
    nflat = hrows * 128
    niter = ept // LANES

    @functools.partial(
        pl.kernel, mesh=mesh,
        out_type=jax.ShapeDtypeStruct((NC * NS, 1, nflat), jnp.float32),
        compiler_params=pltpu.CompilerParams(needs_layout_passes=False),
        scratch_types=[
            pltpu.VMEM((niter, LANES), jnp.int32),
            pltpu.VMEM((nflat,), jnp.float32),
        ],
    )
    def k(dst_hbm, zeros_hbm, out_hbm, dst_v, hist):
        cid = lax.axis_index("c")
        sid = lax.axis_index("s")
        gid = cid * NS + sid
        pltpu.sync_copy(dst_hbm.at[gid], dst_v)
        pltpu.sync_copy(zeros_hbm.at[0, 0], hist)
        ones = jnp.ones((LANES,), jnp.float32)

        def step(i, carry):
            v = dst_v[i, :]
            plsc.addupdate_scatter(hist, [v], ones)
            return carry

        lax.fori_loop(0, niter, step, 0)
        pltpu.sync_copy(hist, out_hbm.at[gid, 0])

    return k


@functools.lru_cache(maxsize=None)
def _make_scatter_kernel(n, d, nchunk, ch):
    mesh = plsc.VectorSubcoreMesh(core_axis_name="c", subcore_axis_name="s",
                                  num_cores=NC, num_subcores=NS)
    rpt = _rows_per_tile(n)
    npad = rpt * NS

    nb = 4
    win = 20

    @functools.partial(
        pl.kernel, mesh=mesh,
        out_type=jax.ShapeDtypeStruct((NC, npad, d), jnp.float32),
        scratch_types=[
            pltpu.VMEM((2, win, ch), jnp.int32),
            pltpu.VMEM((2, win, ch), jnp.int32),
            [pltpu.VMEM((ch, d), jnp.float32)] * nb,
            pltpu.VMEM_SHARED((npad, d), jnp.float32),
            [pltpu.SemaphoreType.DMA] * nb,
            pltpu.SemaphoreType.DMA,
            pltpu.SemaphoreType.DMA,
        ],
    )
    def k(hs_hbm, src_hbm, dst_hbm, zeros_hbm, out_hbm,
          src_w, dst_w, rows, acc, gsems, isem_s, isem_d):
        nwin = nchunk // win
        cid = lax.axis_index("c")
        sid = lax.axis_index("s")
        gid = cid * NS + sid
        pltpu.sync_copy(src_hbm.at[gid, 0], src_w.at[0])
        pltpu.sync_copy(dst_hbm.at[gid, 0], dst_w.at[0])
        pltpu.sync_copy(zeros_hbm.at[pl.ds(sid * rpt, rpt)],
                        acc.at[pl.ds(sid * rpt, rpt)])
        plsc.subcore_barrier()

        for b in range(nb):
            pltpu.async_copy(hs_hbm.at[src_w.at[0, b]], rows[b], gsems[b])

        def wbody(w, carry):
            slot = lax.rem(w, 2)
            nslot = lax.rem(w + 1, 2)
            not_last = w + 1 < nwin

            @pl.when(not_last)
            def _prefetch_idx():
                pltpu.async_copy(src_hbm.at[gid, w + 1], src_w.at[nslot],
                                 isem_s)
                pltpu.async_copy(dst_hbm.at[gid, w + 1], dst_w.at[nslot],
                                 isem_d)

            for k_ in range(win):
                b = k_ % nb
                pltpu.make_async_copy(
                    hs_hbm.at[src_w.at[slot, k_]], rows[b], gsems[b]).wait()
                pltpu.sync_copy(rows[b], acc.at[dst_w.at[slot, k_]], add=True)
                if k_ + nb < win:
                    pltpu.async_copy(hs_hbm.at[src_w.at[slot, k_ + nb]],
                                     rows[b], gsems[b])
                else:
                    if k_ + nb == win:
                        @pl.when(not_last)
                        def _wait_idx():
                            pltpu.make_async_copy(
                                src_hbm.at[gid, 0], src_w.at[nslot],
                                isem_s).wait()
                            pltpu.make_async_copy(
                                dst_hbm.at[gid, 0], dst_w.at[nslot],
                                isem_d).wait()

                    @pl.when(not_last)
                    def _next_win_gather():
                        pltpu.async_copy(
                            hs_hbm.at[src_w.at[nslot, k_ + nb - win]],
                            rows[b], gsems[b])
            return carry

        lax.fori_loop(0, nwin, wbody, 0)
        plsc.subcore_barrier()
        pltpu.sync_copy(acc.at[pl.ds(sid * rpt, rpt)],
                        out_hbm.at[cid, pl.ds(sid * rpt, rpt)])

    return k


def kernel(x, edge_index, batch, W1, b1, gm1, bt1, W2, b2, gm2, bt2,
           W3, b3, gm3, bt3, W4, b4, gm4, bt4, G1w, G1b, G2w, G2b, Wc, bc):
    n, _ = x.shape
    e = edge_index.shape[1]
    dh = W1.shape[0]
    nw = NC * NS
    ept = e // nw
    npad = _rows_per_tile(n) * NS
    win = 20
    nchunk = -(-ept // (CH * win)) * win
    pad = nchunk * CH - ept
    srcT = edge_index[0].reshape(nw, ept)
    dstT = edge_index[1].reshape(nw, ept)
    padsrc = jnp.zeros((nw, pad), jnp.int32)
    paddst = jnp.broadcast_to(
        n + (jnp.arange(pad, dtype=jnp.int32) % (npad - n)), (nw, pad))
    src = jnp.concatenate([srcT, padsrc], 1).reshape(nw, nchunk // win, win, CH)
    dst = jnp.concatenate([dstT, paddst], 1).reshape(nw, nchunk // win, win, CH)
    dst16 = edge_index[1].reshape(nw, ept // LANES, LANES)
    zeros_nd = jnp.zeros((npad, dh), jnp.float32)

    hrows = -(-n // (128 * 8)) * 8
    zeros_flat = jnp.zeros((1, 1, hrows * 128), jnp.float32)
    deg_p = _make_deg_kernel(n, ept)(dst16, zeros_flat)
    dinv2d = _tc(_dinv_body, (hrows, 128), deg_p.reshape(nw, hrows, 128))
    dinv = dinv2d.reshape(hrows * 128, 1)[:n]

    sck = _make_scatter_kernel(n, dh, nchunk, CH)
    layers = ((W1, b1, gm1, bt1), (W2, b2, gm2, bt2),
              (W3, b3, gm3, bt3), (W4, b4, gm4, bt4))
    hs = _tc(_mm_scale_body, (n, dh), x, W1, dinv)
    for i in range(3):
        p = sck(hs, src, dst, zeros_nd)
        b, g, bt = layers[i][1:]
        hs = _tc(_bn_mm_body, (n, dh), p[0, :n], p[1, :n], hs, dinv,
                 b.reshape(1, -1), g.reshape(1, -1), bt.reshape(1, -1),
                 layers[i + 1][0])
    p = sck(hs, src, dst, zeros_nd)
    g2wb = jnp.broadcast_to(G2w, (dh, G2w.shape[1]))
    return _tc(_bn_pool_body, (1, Wc.shape[0]), p[0, :n], p[1, :n], hs, dinv,
               b4.reshape(1, -1), gm4.reshape(1, -1), bt4.reshape(1, -1),
               G1w, G1b.reshape(1, -1), g2wb, Wc, bc.reshape(1, -1))


# speedup vs baseline: 2.9182x; 2.9182x over previous
import functools

import jax
import jax.numpy as jnp
from jax import lax
from jax.experimental import pallas as pl
from jax.experimental.pallas import tpu as pltpu
from jax.experimental.pallas import tpu_sc as plsc

NC = 2
NS = 16
LANES = 16
CH = 125
WIN = 10


def _mm_scale_body(h_ref, w_ref, dinv_ref, o_ref):
    o_ref[...] = lax.dot_general(
        h_ref[...], w_ref[...], (((1,), (1,)), ((), ())),
        preferred_element_type=jnp.float32) * dinv_ref[...]


def _bn_relu(p0, p1, hs, dinv, b, g, bt):
    agg = (p0 + p1 + hs) * dinv + b
    mean = jnp.mean(agg, axis=0, keepdims=True)
    ctr = agg - mean
    var = jnp.mean(ctr * ctr, axis=0, keepdims=True)
    return jnp.maximum(g * ctr * lax.rsqrt(var + 1e-5) + bt, 0.0)


def _bn_mm_body(p0_ref, p1_ref, hs_ref, dinv_ref, b_ref, g_ref, bt_ref,
                w_ref, o_ref):
    h = _bn_relu(p0_ref[...], p1_ref[...], hs_ref[...], dinv_ref[...],
                 b_ref[...], g_ref[...], bt_ref[...])
    o_ref[...] = lax.dot_general(
        h, w_ref[...], (((1,), (1,)), ((), ())),
        preferred_element_type=jnp.float32) * dinv_ref[...]


def _dinv_body(p_ref, o_ref):
    deg = jnp.sum(p_ref[...], axis=0) + 1.0
    o_ref[...] = lax.rsqrt(deg)


def _bn_pool_body(p0_ref, p1_ref, hs_ref, dinv_ref, b_ref, g_ref, bt_ref,
                  g1w_ref, g1b_ref, g2wb_ref, wc_ref, bc_ref, o_ref):
    h = _bn_relu(p0_ref[...], p1_ref[...], hs_ref[...], dinv_ref[...],
                 b_ref[...], g_ref[...], bt_ref[...])
    z = jnp.maximum(
        lax.dot_general(h, g1w_ref[...], (((1,), (1,)), ((), ())),
                        preferred_element_type=jnp.float32) + g1b_ref[...], 0.0)
    gate = lax.dot_general(z, g2wb_ref[...], (((1,), (1,)), ((), ())),
                           preferred_element_type=jnp.float32)
    m = jnp.max(gate, axis=0, keepdims=True)
    e = jnp.exp(gate - m)
    alpha = e / jnp.sum(e, axis=0, keepdims=True)
    pooled = jnp.sum(alpha * h, axis=0, keepdims=True)
    o_ref[...] = lax.dot_general(pooled, wc_ref[...], (((1,), (1,)), ((), ())),
                                 preferred_element_type=jnp.float32) + bc_ref[...]


def _tc(body, shape, *args):
    return pl.pallas_call(
        body, out_shape=jax.ShapeDtypeStruct(shape, jnp.float32))(*args)


def _rows_per_tile(n):
    return -(-n // (NS * 8)) * 8


@functools.lru_cache(maxsize=None)
def _make_deg_kernel(n, ept):
    mesh = plsc.VectorSubcoreMesh(core_axis_name="c", subcore_axis_name="s",
                                  num_cores=NC, num_subcores=NS)
    hrows = -(-n // (128 * 8)) * 8
    nflat = hrows * 128
    niter = ept // LANES

    @functools.partial(
        pl.kernel, mesh=mesh,
        out_type=jax.ShapeDtypeStruct((NC * NS, 1, nflat), jnp.float32),
        compiler_params=pltpu.CompilerParams(needs_layout_passes=False),
        scratch_types=[
            pltpu.VMEM((niter, LANES), jnp.int32),
            pltpu.VMEM((nflat,), jnp.float32),
        ],
    )
    def k(dst_hbm, zeros_hbm, out_hbm, dst_v, hist):
        cid = lax.axis_index("c")
        sid = lax.axis_index("s")
        gid = cid * NS + sid
        pltpu.sync_copy(dst_hbm.at[gid], dst_v)
        pltpu.sync_copy(zeros_hbm.at[0, 0], hist)
        ones = jnp.ones((LANES,), jnp.float32)

        def step(i, carry):
            v = dst_v[i, :]
            plsc.addupdate_scatter(hist, [v], ones)
            return carry

        lax.fori_loop(0, niter, step, 0)
        pltpu.sync_copy(hist, out_hbm.at[gid, 0])

    return k


@functools.lru_cache(maxsize=None)
def _make_scatter_kernel(n, d, nchunk, ch):
    mesh = plsc.VectorSubcoreMesh(core_axis_name="c", subcore_axis_name="s",
                                  num_cores=NC, num_subcores=NS)
    rpt = _rows_per_tile(n)
    npad = rpt * NS

    nb = 2
    win = WIN

    @functools.partial(
        pl.kernel, mesh=mesh,
        out_type=jax.ShapeDtypeStruct((NC, npad, d), jnp.float32),
        scratch_types=[
            pltpu.VMEM((2, win, ch), jnp.int32),
            pltpu.VMEM((2, win, ch), jnp.int32),
            [pltpu.VMEM((ch, d), jnp.float32)] * nb,
            pltpu.VMEM_SHARED((npad, d), jnp.float32),
            [pltpu.SemaphoreType.DMA] * nb,
            pltpu.SemaphoreType.DMA,
            pltpu.SemaphoreType.DMA,
        ],
    )
    def k(hs_hbm, src_hbm, dst_hbm, zeros_hbm, out_hbm,
          src_w, dst_w, rows, acc, gsems, isem_s, isem_d):
        nwin = nchunk // win
        cid = lax.axis_index("c")
        sid = lax.axis_index("s")
        gid = cid * NS + sid
        pltpu.sync_copy(src_hbm.at[gid, 0], src_w.at[0])
        pltpu.sync_copy(dst_hbm.at[gid, 0], dst_w.at[0])
        pltpu.sync_copy(zeros_hbm.at[pl.ds(sid * rpt, rpt)],
                        acc.at[pl.ds(sid * rpt, rpt)])
        plsc.subcore_barrier()

        for b in range(nb):
            pltpu.async_copy(hs_hbm.at[src_w.at[0, b]], rows[b], gsems[b])

        def wbody(w, carry):
            slot = lax.rem(w, 2)
            nslot = lax.rem(w + 1, 2)
            not_last = w + 1 < nwin

            @pl.when(not_last)
            def _prefetch_idx():
                pltpu.async_copy(src_hbm.at[gid, w + 1], src_w.at[nslot],
                                 isem_s)
                pltpu.async_copy(dst_hbm.at[gid, w + 1], dst_w.at[nslot],
                                 isem_d)

            for k_ in range(win):
                b = k_ % nb
                pltpu.make_async_copy(
                    hs_hbm.at[src_w.at[slot, k_]], rows[b], gsems[b]).wait()
                pltpu.sync_copy(rows[b], acc.at[dst_w.at[slot, k_]], add=True)
                if k_ + nb < win:
                    pltpu.async_copy(hs_hbm.at[src_w.at[slot, k_ + nb]],
                                     rows[b], gsems[b])
                else:
                    if k_ + nb == win:
                        @pl.when(not_last)
                        def _wait_idx():
                            pltpu.make_async_copy(
                                src_hbm.at[gid, 0], src_w.at[nslot],
                                isem_s).wait()
                            pltpu.make_async_copy(
                                dst_hbm.at[gid, 0], dst_w.at[nslot],
                                isem_d).wait()

                    @pl.when(not_last)
                    def _next_win_gather():
                        pltpu.async_copy(
                            hs_hbm.at[src_w.at[nslot, k_ + nb - win]],
                            rows[b], gsems[b])
            return carry

        lax.fori_loop(0, nwin, wbody, 0)
        plsc.subcore_barrier()
        pltpu.sync_copy(acc.at[pl.ds(sid * rpt, rpt)],
                        out_hbm.at[cid, pl.ds(sid * rpt, rpt)])

    return k


def kernel(x, edge_index, batch, W1, b1, gm1, bt1, W2, b2, gm2, bt2,
           W3, b3, gm3, bt3, W4, b4, gm4, bt4, G1w, G1b, G2w, G2b, Wc, bc):
    n, _ = x.shape
    e = edge_index.shape[1]
    dh = W1.shape[0]
    nw = NC * NS
    ept = e // nw
    npad = _rows_per_tile(n) * NS
    win = WIN
    nchunk = -(-ept // (CH * win)) * win
    pad = nchunk * CH - ept
    srcT = edge_index[0].reshape(nw, ept)
    dstT = edge_index[1].reshape(nw, ept)
    padsrc = jnp.zeros((nw, pad), jnp.int32)
    paddst = jnp.broadcast_to(
        n + (jnp.arange(pad, dtype=jnp.int32) % (npad - n)), (nw, pad))
    src = jnp.concatenate([srcT, padsrc], 1).reshape(nw, nchunk // win, win, CH)
    dst = jnp.concatenate([dstT, paddst], 1).reshape(nw, nchunk // win, win, CH)
    dst16 = edge_index[1].reshape(nw, ept // LANES, LANES)
    zeros_nd = jnp.zeros((npad, dh), jnp.float32)

    hrows = -(-n // (128 * 8)) * 8
    zeros_flat = jnp.zeros((1, 1, hrows * 128), jnp.float32)
    deg_p = _make_deg_kernel(n, ept)(dst16, zeros_flat)
    dinv2d = _tc(_dinv_body, (hrows, 128), deg_p.reshape(nw, hrows, 128))
    dinv = dinv2d.reshape(hrows * 128, 1)[:n]

    sck = _make_scatter_kernel(n, dh, nchunk, CH)
    layers = ((W1, b1, gm1, bt1), (W2, b2, gm2, bt2),
              (W3, b3, gm3, bt3), (W4, b4, gm4, bt4))
    hs = _tc(_mm_scale_body, (n, dh), x, W1, dinv)
    for i in range(3):
        p = sck(hs, src, dst, zeros_nd)
        b, g, bt = layers[i][1:]
        hs = _tc(_bn_mm_body, (n, dh), p[0, :n], p[1, :n], hs, dinv,
                 b.reshape(1, -1), g.reshape(1, -1), bt.reshape(1, -1),
                 layers[i + 1][0])
    p = sck(hs, src, dst, zeros_nd)
    g2wb = jnp.broadcast_to(G2w, (dh, G2w.shape[1]))
    return _tc(_bn_pool_body, (1, Wc.shape[0]), p[0, :n], p[1, :n], hs, dinv,
               b4.reshape(1, -1), gm4.reshape(1, -1), bt4.reshape(1, -1),
               G1w, G1b.reshape(1, -1), g2wb, Wc, bc.reshape(1, -1))


# speedup vs baseline: 2.9466x; 1.0097x over previous
import functools

import jax
import jax.numpy as jnp
from jax import lax
from jax.experimental import pallas as pl
from jax.experimental.pallas import tpu as pltpu
from jax.experimental.pallas import tpu_sc as plsc

NC = 2
NS = 16
LANES = 16
CH = 125
WIN = 10


def _mm_scale_body(h_ref, w_ref, dinv_ref, o_ref):
    o_ref[...] = lax.dot_general(
        h_ref[...], w_ref[...], (((1,), (1,)), ((), ())),
        preferred_element_type=jnp.float32) * dinv_ref[...]


def _bn_relu(p0, p1, dinv, b, g, bt):
    agg = (p0 + p1) * dinv + b
    mean = jnp.mean(agg, axis=0, keepdims=True)
    ctr = agg - mean
    var = jnp.mean(ctr * ctr, axis=0, keepdims=True)
    return jnp.maximum(g * ctr * lax.rsqrt(var + 1e-5) + bt, 0.0)


def _bn_mm_body(p0_ref, p1_ref, dinv_ref, b_ref, g_ref, bt_ref,
                w_ref, o_ref):
    h = _bn_relu(p0_ref[...], p1_ref[...], dinv_ref[...],
                 b_ref[...], g_ref[...], bt_ref[...])
    o_ref[...] = lax.dot_general(
        h, w_ref[...], (((1,), (1,)), ((), ())),
        preferred_element_type=jnp.float32) * dinv_ref[...]


def _dinv_body(p_ref, o_ref):
    deg = jnp.sum(p_ref[...], axis=0) + 1.0
    o_ref[...] = lax.rsqrt(deg)


def _bn_pool_body(p0_ref, p1_ref, dinv_ref, b_ref, g_ref, bt_ref,
                  g1w_ref, g1b_ref, g2wb_ref, wc_ref, bc_ref, o_ref):
    h = _bn_relu(p0_ref[...], p1_ref[...], dinv_ref[...],
                 b_ref[...], g_ref[...], bt_ref[...])
    z = jnp.maximum(
        lax.dot_general(h, g1w_ref[...], (((1,), (1,)), ((), ())),
                        preferred_element_type=jnp.float32) + g1b_ref[...], 0.0)
    gate = lax.dot_general(z, g2wb_ref[...], (((1,), (1,)), ((), ())),
                           preferred_element_type=jnp.float32)
    m = jnp.max(gate, axis=0, keepdims=True)
    e = jnp.exp(gate - m)
    alpha = e / jnp.sum(e, axis=0, keepdims=True)
    pooled = jnp.sum(alpha * h, axis=0, keepdims=True)
    o_ref[...] = lax.dot_general(pooled, wc_ref[...], (((1,), (1,)), ((), ())),
                                 preferred_element_type=jnp.float32) + bc_ref[...]


def _tc(body, shape, *args):
    return pl.pallas_call(
        body, out_shape=jax.ShapeDtypeStruct(shape, jnp.float32))(*args)


def _rows_per_tile(n):
    return -(-n // (NS * 8)) * 8


@functools.lru_cache(maxsize=None)
def _make_deg_kernel(n, ept):
    mesh = plsc.VectorSubcoreMesh(core_axis_name="c", subcore_axis_name="s",
                                  num_cores=NC, num_subcores=NS)
    hrows = -(-n // (128 * 8)) * 8
    nflat = hrows * 128
    niter = ept // LANES

    @functools.partial(
        pl.kernel, mesh=mesh,
        out_type=jax.ShapeDtypeStruct((NC * NS, 1, nflat), jnp.float32),
        compiler_params=pltpu.CompilerParams(needs_layout_passes=False),
        scratch_types=[
            pltpu.VMEM((niter, LANES), jnp.int32),
            pltpu.VMEM((nflat,), jnp.float32),
        ],
    )
    def k(dst_hbm, zeros_hbm, out_hbm, dst_v, hist):
        cid = lax.axis_index("c")
        sid = lax.axis_index("s")
        gid = cid * NS + sid
        pltpu.sync_copy(dst_hbm.at[gid], dst_v)
        pltpu.sync_copy(zeros_hbm.at[0, 0], hist)
        ones = jnp.ones((LANES,), jnp.float32)

        def step(i, carry):
            v = dst_v[i, :]
            plsc.addupdate_scatter(hist, [v], ones)
            return carry

        lax.fori_loop(0, niter, step, 0)
        pltpu.sync_copy(hist, out_hbm.at[gid, 0])

    return k


@functools.lru_cache(maxsize=None)
def _make_scatter_kernel(n, d, nchunk, ch):
    mesh = plsc.VectorSubcoreMesh(core_axis_name="c", subcore_axis_name="s",
                                  num_cores=NC, num_subcores=NS)
    rpt = _rows_per_tile(n)
    npad = rpt * NS

    nb = 2
    win = WIN

    @functools.partial(
        pl.kernel, mesh=mesh,
        out_type=jax.ShapeDtypeStruct((NC, npad, d), jnp.float32),
        scratch_types=[
            pltpu.VMEM((2, win, ch), jnp.int32),
            pltpu.VMEM((2, win, ch), jnp.int32),
            [pltpu.VMEM((ch, d), jnp.float32)] * nb,
            pltpu.VMEM_SHARED((npad, d), jnp.float32),
            [pltpu.SemaphoreType.DMA] * nb,
            pltpu.SemaphoreType.DMA,
            pltpu.SemaphoreType.DMA,
        ],
    )
    def k(hs_hbm, src_hbm, dst_hbm, zeros_hbm, out_hbm,
          src_w, dst_w, rows, acc, gsems, isem_s, isem_d):
        nwin = nchunk // win
        cid = lax.axis_index("c")
        sid = lax.axis_index("s")
        gid = cid * NS + sid
        pltpu.sync_copy(src_hbm.at[gid, 0], src_w.at[0])
        pltpu.sync_copy(dst_hbm.at[gid, 0], dst_w.at[0])
        tail = npad - n
        full = min(rpt, n - 15 * rpt)

        @pl.when(jnp.logical_and(cid == 0, sid < NS - 1))
        def _seed_hs():
            pltpu.sync_copy(hs_hbm.at[pl.ds(sid * rpt, rpt)],
                            acc.at[pl.ds(sid * rpt, rpt)])

        @pl.when(jnp.logical_and(cid == 0, sid == NS - 1))
        def _seed_hs_tail():
            pltpu.sync_copy(hs_hbm.at[pl.ds((NS - 1) * rpt, full)],
                            acc.at[pl.ds((NS - 1) * rpt, full)])
            pltpu.sync_copy(zeros_hbm.at[pl.ds(0, tail)],
                            acc.at[pl.ds(n, tail)])

        @pl.when(cid == 1)
        def _seed_zero():
            pltpu.sync_copy(zeros_hbm.at[pl.ds(sid * rpt, rpt)],
                            acc.at[pl.ds(sid * rpt, rpt)])

        plsc.subcore_barrier()

        for b in range(nb):
            pltpu.async_copy(hs_hbm.at[src_w.at[0, b]], rows[b], gsems[b])

        def wbody(w, carry):
            slot = lax.rem(w, 2)
            nslot = lax.rem(w + 1, 2)
            not_last = w + 1 < nwin

            @pl.when(not_last)
            def _prefetch_idx():
                pltpu.async_copy(src_hbm.at[gid, w + 1], src_w.at[nslot],
                                 isem_s)
                pltpu.async_copy(dst_hbm.at[gid, w + 1], dst_w.at[nslot],
                                 isem_d)

            for k_ in range(win):
                b = k_ % nb
                pltpu.make_async_copy(
                    hs_hbm.at[src_w.at[slot, k_]], rows[b], gsems[b]).wait()
                pltpu.sync_copy(rows[b], acc.at[dst_w.at[slot, k_]], add=True)
                if k_ + nb < win:
                    pltpu.async_copy(hs_hbm.at[src_w.at[slot, k_ + nb]],
                                     rows[b], gsems[b])
                else:
                    if k_ + nb == win:
                        @pl.when(not_last)
                        def _wait_idx():
                            pltpu.make_async_copy(
                                src_hbm.at[gid, 0], src_w.at[nslot],
                                isem_s).wait()
                            pltpu.make_async_copy(
                                dst_hbm.at[gid, 0], dst_w.at[nslot],
                                isem_d).wait()

                    @pl.when(not_last)
                    def _next_win_gather():
                        pltpu.async_copy(
                            hs_hbm.at[src_w.at[nslot, k_ + nb - win]],
                            rows[b], gsems[b])
            return carry

        lax.fori_loop(0, nwin, wbody, 0)
        plsc.subcore_barrier()
        pltpu.sync_copy(acc.at[pl.ds(sid * rpt, rpt)],
                        out_hbm.at[cid, pl.ds(sid * rpt, rpt)])

    return k


def kernel(x, edge_index, batch, W1, b1, gm1, bt1, W2, b2, gm2, bt2,
           W3, b3, gm3, bt3, W4, b4, gm4, bt4, G1w, G1b, G2w, G2b, Wc, bc):
    n, _ = x.shape
    e = edge_index.shape[1]
    dh = W1.shape[0]
    nw = NC * NS
    ept = e // nw
    npad = _rows_per_tile(n) * NS
    win = WIN
    nchunk = -(-ept // (CH * win)) * win
    pad = nchunk * CH - ept
    srcT = edge_index[0].reshape(nw, ept)
    dstT = edge_index[1].reshape(nw, ept)
    padsrc = jnp.zeros((nw, pad), jnp.int32)
    paddst = jnp.broadcast_to(
        n + (jnp.arange(pad, dtype=jnp.int32) % (npad - n)), (nw, pad))
    src = jnp.concatenate([srcT, padsrc], 1).reshape(nw, nchunk // win, win, CH)
    dst = jnp.concatenate([dstT, paddst], 1).reshape(nw, nchunk // win, win, CH)
    dst16 = edge_index[1].reshape(nw, ept // LANES, LANES)
    zeros_nd = jnp.zeros((npad, dh), jnp.float32)

    hrows = -(-n // (128 * 8)) * 8
    zeros_flat = jnp.zeros((1, 1, hrows * 128), jnp.float32)
    deg_p = _make_deg_kernel(n, ept)(dst16, zeros_flat)
    dinv2d = _tc(_dinv_body, (hrows, 128), deg_p.reshape(nw, hrows, 128))
    dinv = dinv2d.reshape(hrows * 128, 1)[:n]

    sck = _make_scatter_kernel(n, dh, nchunk, CH)
    layers = ((W1, b1, gm1, bt1), (W2, b2, gm2, bt2),
              (W3, b3, gm3, bt3), (W4, b4, gm4, bt4))
    hs = _tc(_mm_scale_body, (n, dh), x, W1, dinv)
    for i in range(3):
        p = sck(hs, src, dst, zeros_nd)
        b, g, bt = layers[i][1:]
        hs = _tc(_bn_mm_body, (n, dh), p[0, :n], p[1, :n], dinv,
                 b.reshape(1, -1), g.reshape(1, -1), bt.reshape(1, -1),
                 layers[i + 1][0])
    p = sck(hs, src, dst, zeros_nd)
    g2wb = jnp.broadcast_to(G2w, (dh, G2w.shape[1]))
    return _tc(_bn_pool_body, (1, Wc.shape[0]), p[0, :n], p[1, :n], dinv,
               b4.reshape(1, -1), gm4.reshape(1, -1), bt4.reshape(1, -1),
               G1w, G1b.reshape(1, -1), g2wb, Wc, bc.reshape(1, -1))
